# Initial kernel scaffold; baseline (speedup 1.0000x reference)
import jax
import jax.numpy as jnp
from jax.experimental import pallas as pl


def kernel(X, ppr_scores, ppr_idx, W1, W2, W3, W4):
    raise NotImplementedError("write your pallas kernel here")


# speedup vs baseline: 2.0718x; 2.0718x over previous
import functools

import jax
import jax.numpy as jnp
from jax import lax
from jax.experimental import pallas as pl
from jax.experimental.pallas import tpu as pltpu
from jax.experimental.pallas import tpu_sc as plsc

_N_PPR = 320000
_D = 128
_H = 128
_NCLS = 64
_NB = 10000

_CH = 128
_NCHUNK = _N_PPR // _CH
_NW = 32
_CBASE = _NCHUNK // _NW
_CEXTRA = _NCHUNK - _CBASE * _NW
_NBP = 10240
_ROWS_PER_SUB = _NBP // 16

_BLK = 2000


def _emb_body(x_ref, s_ref, w1_ref, w2_ref, o_ref):
    h = jnp.maximum(
        jnp.dot(x_ref[...], w1_ref[...], preferred_element_type=jnp.float32), 0.0)
    e = jnp.dot(h, w2_ref[...], preferred_element_type=jnp.float32)
    o_ref[...] = e * s_ref[...]


def _emb_call(X, scores_col, W1, W2):
    return pl.pallas_call(
        _emb_body,
        grid=(_N_PPR // _BLK,),
        in_specs=[
            pl.BlockSpec((_BLK, _D), lambda i: (i, 0)),
            pl.BlockSpec((_BLK, 1), lambda i: (i, 0)),
            pl.BlockSpec((_D, _H), lambda i: (0, 0)),
            pl.BlockSpec((_H, _H), lambda i: (0, 0)),
        ],
        out_specs=pl.BlockSpec((_BLK, _H), lambda i: (i, 0)),
        out_shape=jax.ShapeDtypeStruct((_N_PPR, _H), jnp.float32),
    )(X, scores_col, W1, W2)


def _scatter_sc(emb, idx1d, zrows):
    mesh = plsc.VectorSubcoreMesh(core_axis_name="c", subcore_axis_name="s")

    @functools.partial(
        pl.kernel,
        mesh=mesh,
        out_type=jax.ShapeDtypeStruct((2 * _NBP, _H), jnp.float32),
        scratch_types=[
            pltpu.VMEM((_CH,), jnp.int32),
            pltpu.VMEM((_CH, _H), jnp.float32),
            pltpu.VMEM_SHARED((_NBP, _H), jnp.float32),
        ],
    )
    def k(emb_hbm, idx_hbm, z_hbm, out_hbm, idx_v, rows_v, acc):
        c = lax.axis_index("c")
        s = lax.axis_index("s")
        wid = s * 2 + c
        pltpu.sync_copy(z_hbm, acc.at[pl.ds(s * _ROWS_PER_SUB, _ROWS_PER_SUB)])
        plsc.subcore_barrier()
        nch = _CBASE + jnp.where(wid < _CEXTRA, 1, 0)
        base = wid * _CBASE + jnp.minimum(wid, _CEXTRA)

        def body(j, carry):
            ch = base + j
            pltpu.sync_copy(idx_hbm.at[pl.ds(ch * _CH, _CH)], idx_v)
            pltpu.sync_copy(emb_hbm.at[pl.ds(ch * _CH, _CH)], rows_v)
            pltpu.sync_copy(rows_v, acc.at[idx_v], add=True)
            return carry

        lax.fori_loop(0, nch, body, 0)
        plsc.subcore_barrier()
        pltpu.sync_copy(
            acc.at[pl.ds(s * _ROWS_PER_SUB, _ROWS_PER_SUB)],
            out_hbm.at[pl.ds(c * _NBP + s * _ROWS_PER_SUB, _ROWS_PER_SUB)])

    return k(emb, idx1d, zrows)


def _mlp_body(p_ref, w3_ref, w4_ref, o_ref):
    p = p_ref[0:_NB, :] + p_ref[_NBP:_NBP + _NB, :]
    h = jnp.maximum(
        jnp.dot(p, w3_ref[...], preferred_element_type=jnp.float32), 0.0)
    o_ref[...] = jnp.dot(h, w4_ref[...], preferred_element_type=jnp.float32)


def _mlp_call(partials, W3, W4):
    return pl.pallas_call(
        _mlp_body,
        out_shape=jax.ShapeDtypeStruct((_NB, _NCLS), jnp.float32),
    )(partials, W3, W4)


def kernel(X, ppr_scores, ppr_idx, W1, W2, W3, W4):
    emb = _emb_call(X, ppr_scores.reshape(_N_PPR, 1), W1, W2)
    zrows = jnp.zeros((_ROWS_PER_SUB, _H), jnp.float32)
    partials = _scatter_sc(emb, ppr_idx, zrows)
    return _mlp_call(partials, W3, W4)


# speedup vs baseline: 2.4995x; 1.2064x over previous
import functools

import jax
import jax.numpy as jnp
from jax import lax
from jax.experimental import pallas as pl
from jax.experimental.pallas import tpu as pltpu
from jax.experimental.pallas import tpu_sc as plsc

_N_PPR = 320000
_D = 128
_H = 128
_NCLS = 64
_NB = 10000

_CH = 128
_NCHUNK = _N_PPR // _CH
_NW = 32
_CBASE = _NCHUNK // _NW
_CEXTRA = _NCHUNK - _CBASE * _NW
_NBP = 10240
_ROWS_PER_SUB = _NBP // 16

_BLK = 2000


def _emb_body(x_ref, s_ref, w1_ref, w2_ref, o_ref):
    h = jnp.maximum(
        jnp.dot(x_ref[...], w1_ref[...], preferred_element_type=jnp.float32), 0.0)
    e = jnp.dot(h, w2_ref[...], preferred_element_type=jnp.float32)
    o_ref[...] = e * s_ref[...]


def _emb_call(X, scores_col, W1, W2):
    return pl.pallas_call(
        _emb_body,
        grid=(_N_PPR // _BLK,),
        in_specs=[
            pl.BlockSpec((_BLK, _D), lambda i: (i, 0)),
            pl.BlockSpec((_BLK, 1), lambda i: (i, 0)),
            pl.BlockSpec((_D, _H), lambda i: (0, 0)),
            pl.BlockSpec((_H, _H), lambda i: (0, 0)),
        ],
        out_specs=pl.BlockSpec((_BLK, _H), lambda i: (i, 0)),
        out_shape=jax.ShapeDtypeStruct((_N_PPR, _H), jnp.float32),
    )(X, scores_col, W1, W2)


def _scatter_sc(emb, idx1d, zrows):
    mesh = plsc.VectorSubcoreMesh(core_axis_name="c", subcore_axis_name="s")

    @functools.partial(
        pl.kernel,
        mesh=mesh,
        out_type=jax.ShapeDtypeStruct((2 * _NBP, _H), jnp.float32),
        scratch_types=[
            pltpu.VMEM((2, _CH), jnp.int32),
            pltpu.VMEM((2, _CH, _H), jnp.float32),
            pltpu.VMEM_SHARED((_NBP, _H), jnp.float32),
            pltpu.SemaphoreType.DMA,
            pltpu.SemaphoreType.DMA,
            pltpu.SemaphoreType.DMA,
            pltpu.SemaphoreType.DMA,
        ],
    )
    def k(emb_hbm, idx_hbm, z_hbm, out_hbm, idx_v, rows_v, acc,
          isem0, isem1, rsem0, rsem1):
        c = lax.axis_index("c")
        s = lax.axis_index("s")
        wid = s * 2 + c
        isem = (isem0, isem1)
        rsem = (rsem0, rsem1)
        pltpu.sync_copy(z_hbm, acc.at[pl.ds(s * _ROWS_PER_SUB, _ROWS_PER_SUB)])
        plsc.subcore_barrier()
        nch = _CBASE + jnp.where(wid < _CEXTRA, 1, 0)
        base = wid * _CBASE + jnp.minimum(wid, _CEXTRA)

        def _copies(j, b):
            ch = base + j
            return (
                pltpu.make_async_copy(
                    idx_hbm.at[pl.ds(ch * _CH, _CH)], idx_v.at[b], isem[b]),
                pltpu.make_async_copy(
                    emb_hbm.at[pl.ds(ch * _CH, _CH)], rows_v.at[b], rsem[b]),
            )

        for cp in _copies(0, 0):
            cp.start()
        for cp in _copies(1, 1):
            cp.start()

        def body(g, carry):
            for b in range(2):
                j = 2 * g + b

                @pl.when(j < nch)
                def _():
                    for cp in _copies(j, b):
                        cp.wait()
                    pltpu.sync_copy(rows_v.at[b], acc.at[idx_v.at[b]], add=True)

                    @pl.when(j + 2 < nch)
                    def _():
                        for cp in _copies(j + 2, b):
                            cp.start()
            return carry

        lax.fori_loop(0, (_CBASE + 2) // 2, body, 0)
        plsc.subcore_barrier()
        pltpu.sync_copy(
            acc.at[pl.ds(s * _ROWS_PER_SUB, _ROWS_PER_SUB)],
            out_hbm.at[pl.ds(c * _NBP + s * _ROWS_PER_SUB, _ROWS_PER_SUB)])

    return k(emb, idx1d, zrows)


def _mlp_body(p_ref, w3_ref, w4_ref, o_ref):
    p = p_ref[0:_NB, :] + p_ref[_NBP:_NBP + _NB, :]
    h = jnp.maximum(
        jnp.dot(p, w3_ref[...], preferred_element_type=jnp.float32), 0.0)
    o_ref[...] = jnp.dot(h, w4_ref[...], preferred_element_type=jnp.float32)


def _mlp_call(partials, W3, W4):
    return pl.pallas_call(
        _mlp_body,
        out_shape=jax.ShapeDtypeStruct((_NB, _NCLS), jnp.float32),
    )(partials, W3, W4)


def kernel(X, ppr_scores, ppr_idx, W1, W2, W3, W4):
    emb = _emb_call(X, ppr_scores.reshape(_N_PPR, 1), W1, W2)
    zrows = jnp.zeros((_ROWS_PER_SUB, _H), jnp.float32)
    partials = _scatter_sc(emb, ppr_idx, zrows)
    return _mlp_call(partials, W3, W4)


# speedup vs baseline: 2.8322x; 1.1331x over previous
import functools

import jax
import jax.numpy as jnp
from jax import lax
from jax.experimental import pallas as pl
from jax.experimental.pallas import tpu as pltpu
from jax.experimental.pallas import tpu_sc as plsc

_N_PPR = 320000
_D = 128
_H = 128
_NCLS = 64
_NB = 10000

_CH = 128
_NCHUNK = _N_PPR // _CH
_NW = 32
_CBASE = _NCHUNK // _NW
_CEXTRA = _NCHUNK - _CBASE * _NW
_NBP = 10240
_ROWS_PER_SUB = _NBP // 16

_BLK = 4000


def _emb_body(x_ref, s_ref, w1_ref, w2_ref, o_ref):
    h = jnp.maximum(
        jnp.dot(x_ref[...], w1_ref[...], preferred_element_type=jnp.float32), 0.0)
    e = jnp.dot(h, w2_ref[...], preferred_element_type=jnp.float32)
    o_ref[...] = e * s_ref[...]


def _emb_call(X, scores_col, W1, W2):
    return pl.pallas_call(
        _emb_body,
        grid=(_N_PPR // _BLK,),
        in_specs=[
            pl.BlockSpec((_BLK, _D), lambda i: (i, 0)),
            pl.BlockSpec((_BLK, 1), lambda i: (i, 0)),
            pl.BlockSpec((_D, _H), lambda i: (0, 0)),
            pl.BlockSpec((_H, _H), lambda i: (0, 0)),
        ],
        out_specs=pl.BlockSpec((_BLK, _H), lambda i: (i, 0)),
        out_shape=jax.ShapeDtypeStruct((_N_PPR, _H), jnp.float32),
    )(X, scores_col, W1, W2)


def _scatter_sc(emb, idx1d, zrows):
    mesh = plsc.VectorSubcoreMesh(core_axis_name="c", subcore_axis_name="s")

    @functools.partial(
        pl.kernel,
        mesh=mesh,
        out_type=jax.ShapeDtypeStruct((2 * _NBP, _H), jnp.float32),
        scratch_types=[
            pltpu.VMEM((2, _CH), jnp.int32),
            pltpu.VMEM((2, _CH, _H), jnp.float32),
            pltpu.VMEM_SHARED((_NBP, _H), jnp.float32),
            pltpu.SemaphoreType.DMA,
            pltpu.SemaphoreType.DMA,
            pltpu.SemaphoreType.DMA,
            pltpu.SemaphoreType.DMA,
        ],
    )
    def k(emb_hbm, idx_hbm, z_hbm, out_hbm, idx_v, rows_v, acc,
          isem0, isem1, rsem0, rsem1):
        c = lax.axis_index("c")
        s = lax.axis_index("s")
        wid = s * 2 + c
        isem = (isem0, isem1)
        rsem = (rsem0, rsem1)
        pltpu.sync_copy(z_hbm, acc.at[pl.ds(s * _ROWS_PER_SUB, _ROWS_PER_SUB)])
        plsc.subcore_barrier()
        nch = _CBASE + jnp.where(wid < _CEXTRA, 1, 0)
        base = wid * _CBASE + jnp.minimum(wid, _CEXTRA)

        def _copies(j, b):
            ch = base + j
            return (
                pltpu.make_async_copy(
                    idx_hbm.at[pl.ds(ch * _CH, _CH)], idx_v.at[b], isem[b]),
                pltpu.make_async_copy(
                    emb_hbm.at[pl.ds(ch * _CH, _CH)], rows_v.at[b], rsem[b]),
            )

        for cp in _copies(0, 0):
            cp.start()
        for cp in _copies(1, 1):
            cp.start()

        def body(g, carry):
            for b in range(2):
                j = 2 * g + b

                @pl.when(j < nch)
                def _():
                    for cp in _copies(j, b):
                        cp.wait()
                    pltpu.sync_copy(rows_v.at[b], acc.at[idx_v.at[b]], add=True)

                    @pl.when(j + 2 < nch)
                    def _():
                        for cp in _copies(j + 2, b):
                            cp.start()
            return carry

        lax.fori_loop(0, (_CBASE + 2) // 2, body, 0)
        plsc.subcore_barrier()
        pltpu.sync_copy(
            acc.at[pl.ds(s * _ROWS_PER_SUB, _ROWS_PER_SUB)],
            out_hbm.at[pl.ds(c * _NBP + s * _ROWS_PER_SUB, _ROWS_PER_SUB)])

    return k(emb, idx1d, zrows)


def _mlp_body(p_ref, w3_ref, w4_ref, o_ref):
    p = p_ref[0:_NB, :] + p_ref[_NBP:_NBP + _NB, :]
    h = jnp.maximum(
        jnp.dot(p, w3_ref[...], preferred_element_type=jnp.float32), 0.0)
    o_ref[...] = jnp.dot(h, w4_ref[...], preferred_element_type=jnp.float32)


def _mlp_call(partials, W3, W4):
    return pl.pallas_call(
        _mlp_body,
        out_shape=jax.ShapeDtypeStruct((_NB, _NCLS), jnp.float32),
    )(partials, W3, W4)


def kernel(X, ppr_scores, ppr_idx, W1, W2, W3, W4):
    emb = _emb_call(X, ppr_scores.reshape(_N_PPR, 1), W1, W2)
    zrows = jnp.zeros((_ROWS_PER_SUB, _H), jnp.float32)
    partials = _scatter_sc(emb, ppr_idx, zrows)
    return _mlp_call(partials, W3, W4)


# speedup vs baseline: 4.3876x; 1.5492x over previous
import functools

import jax
import jax.numpy as jnp
from jax import lax
from jax.experimental import pallas as pl
from jax.experimental.pallas import tpu as pltpu
from jax.experimental.pallas import tpu_sc as plsc

_N_PPR = 320000
_D = 128
_H = 128
_NCLS = 64
_NB = 10000

_CH = 128
_NCHUNK = _N_PPR // _CH
_NW = 32
_CBASE = _NCHUNK // _NW
_CEXTRA = _NCHUNK - _CBASE * _NW
_NBP = 10240
_ROWS_PER_SUB = _NBP // 16

_BLK = 6400


_NSTEP = _N_PPR // _BLK
_NSLAB = 2
_SLAB_ROWS = _N_PPR // _NSLAB
_SLAB_STEPS = _NSTEP // _NSLAB
_SLAB_CHUNKS = _NCHUNK // _NSLAB


def _make_emb_body(slab):
    def _emb_body(x_ref, s_ref, w1_ref, w2_ref, o_ref):
        h = jnp.maximum(
            jnp.dot(x_ref[...], w1_ref[...], preferred_element_type=jnp.float32), 0.0)
        e = jnp.dot(h, w2_ref[...], preferred_element_type=jnp.float32)
        i = pl.program_id(0) + slab * _SLAB_STEPS
        oh = (lax.broadcasted_iota(jnp.int32, (_NSTEP, _H), 0) == i)
        s_bcast = jnp.dot(s_ref[...], oh.astype(jnp.float32),
                          preferred_element_type=jnp.float32)
        o_ref[...] = e * s_bcast
    return _emb_body


def _emb_call(X, scores_cm, W1, W2, slab):
    return pl.pallas_call(
        _make_emb_body(slab),
        grid=(_SLAB_STEPS,),
        in_specs=[
            pl.BlockSpec((_BLK, _D), lambda i: (i + slab * _SLAB_STEPS, 0)),
            pl.BlockSpec((_BLK, _NSTEP), lambda i: (0, 0)),
            pl.BlockSpec((_D, _H), lambda i: (0, 0)),
            pl.BlockSpec((_H, _H), lambda i: (0, 0)),
        ],
        out_specs=pl.BlockSpec((_BLK, _H), lambda i: (i, 0)),
        out_shape=jax.ShapeDtypeStruct((_SLAB_ROWS, _H), jnp.float32),
    )(X, scores_cm, W1, W2)


_SCBASE = _SLAB_CHUNKS // _NW
_SCEXTRA = _SLAB_CHUNKS - _SCBASE * _NW


def _scatter_sc(emb, idx1d, zrows, slab):
    mesh = plsc.VectorSubcoreMesh(core_axis_name="c", subcore_axis_name="s")
    idx_off = slab * _SLAB_CHUNKS * _CH

    @functools.partial(
        pl.kernel,
        mesh=mesh,
        out_type=jax.ShapeDtypeStruct((2 * _NBP, _H), jnp.float32),
        scratch_types=[
            pltpu.VMEM((2, _CH), jnp.int32),
            pltpu.VMEM((2, _CH, _H), jnp.float32),
            pltpu.VMEM_SHARED((_NBP, _H), jnp.float32),
            pltpu.SemaphoreType.DMA,
            pltpu.SemaphoreType.DMA,
            pltpu.SemaphoreType.DMA,
            pltpu.SemaphoreType.DMA,
        ],
    )
    def k(emb_hbm, idx_hbm, z_hbm, out_hbm, idx_v, rows_v, acc,
          isem0, isem1, rsem0, rsem1):
        c = lax.axis_index("c")
        s = lax.axis_index("s")
        wid = s * 2 + c
        isem = (isem0, isem1)
        rsem = (rsem0, rsem1)
        pltpu.sync_copy(z_hbm, acc.at[pl.ds(s * _ROWS_PER_SUB, _ROWS_PER_SUB)])
        plsc.subcore_barrier()
        nch = _SCBASE + jnp.where(wid < _SCEXTRA, 1, 0)
        base = wid * _SCBASE + jnp.minimum(wid, _SCEXTRA)

        def _copies(j, b):
            ch = base + j
            return (
                pltpu.make_async_copy(
                    idx_hbm.at[pl.ds(idx_off + ch * _CH, _CH)], idx_v.at[b], isem[b]),
                pltpu.make_async_copy(
                    emb_hbm.at[pl.ds(ch * _CH, _CH)], rows_v.at[b], rsem[b]),
            )

        for cp in _copies(0, 0):
            cp.start()
        for cp in _copies(1, 1):
            cp.start()

        def body(g, carry):
            for b in range(2):
                j = 2 * g + b

                @pl.when(j < nch)
                def _():
                    for cp in _copies(j, b):
                        cp.wait()
                    pltpu.sync_copy(rows_v.at[b], acc.at[idx_v.at[b]], add=True)

                    @pl.when(j + 2 < nch)
                    def _():
                        for cp in _copies(j + 2, b):
                            cp.start()
            return carry

        lax.fori_loop(0, (_SCBASE + 2) // 2, body, 0)
        plsc.subcore_barrier()
        pl.delay(2000)
        plsc.subcore_barrier()
        pltpu.sync_copy(
            acc.at[pl.ds(s * _ROWS_PER_SUB, _ROWS_PER_SUB)],
            out_hbm.at[pl.ds(c * _NBP + s * _ROWS_PER_SUB, _ROWS_PER_SUB)])

    return k(emb, idx1d, zrows)


def _mlp_body(*refs):
    p_refs, (w3_ref, w4_ref, o_ref) = refs[:_NSLAB], refs[_NSLAB:]
    p = p_refs[0][0:_NB, :] + p_refs[0][_NBP:_NBP + _NB, :]
    for pr in p_refs[1:]:
        p = p + pr[0:_NB, :] + pr[_NBP:_NBP + _NB, :]
    h = jnp.maximum(
        jnp.dot(p, w3_ref[...], preferred_element_type=jnp.float32), 0.0)
    o_ref[...] = jnp.dot(h, w4_ref[...], preferred_element_type=jnp.float32)


def _mlp_call(partials, W3, W4):
    return pl.pallas_call(
        _mlp_body,
        out_shape=jax.ShapeDtypeStruct((_NB, _NCLS), jnp.float32),
    )(*partials, W3, W4)


def kernel(X, ppr_scores, ppr_idx, W1, W2, W3, W4):
    scores_cm = ppr_scores.reshape(_NSTEP, _BLK).T
    zrows = jnp.zeros((_ROWS_PER_SUB, _H), jnp.float32)
    partials = []
    for slab in range(_NSLAB):
        emb = _emb_call(X, scores_cm, W1, W2, slab)
        partials.append(_scatter_sc(emb, ppr_idx, zrows, slab))
    return _mlp_call(partials, W3, W4)
